# initial kernel scaffold (unmeasured)
import functools

import jax
import jax.numpy as jnp
from jax import lax
from jax.experimental import pallas as pl
from jax.experimental.pallas import tpu as pltpu

N_DEV = 4
SQ = 2048
SKV = 2048
HQ_LOCAL = 8
DH = 128
DMODEL = 1024
QBLK = 512
N_QB = SQ // QBLK
CHUNK = SQ // N_DEV
SCALE = 0.08838834764831843
LOCAL_WINDOW = 128
GLOBAL_TOKENS = 32


def _body(x_ref, wq_ref, k_ref, v_ref, wo_ref, out_ref,
          partial_ref, rs_recv_ref,
          rs_send_sems, rs_recv_sems, ag_send_sems, ag_recv_sems):
    h = pl.program_id(0)
    qb = pl.program_id(1)

    q = lax.dot_general(
        x_ref[...], wq_ref[...],
        (((1,), (0,)), ((), ())), preferred_element_type=jnp.float32)
    k = k_ref[:, 0, :]
    v = v_ref[:, 0, :]
    s = lax.dot_general(
        q, k, (((1,), (1,)), ((), ())),
        preferred_element_type=jnp.float32) * SCALE

    qi = qb * QBLK + lax.broadcasted_iota(jnp.int32, (QBLK, SKV), 0)
    ki = lax.broadcasted_iota(jnp.int32, (QBLK, SKV), 1)
    local = jnp.abs(qi - ki) <= LOCAL_WINDOW
    glob = (ki < GLOBAL_TOKENS) | (qi < GLOBAL_TOKENS)
    mask = local | glob
    s = jnp.where(mask, s, -1e9)

    m = jnp.max(s, axis=1, keepdims=True)
    e = jnp.exp(s - m)
    denom = jnp.sum(e, axis=1, keepdims=True)
    w = e / denom

    ctx = lax.dot_general(
        w, v, (((1,), (0,)), ((), ())), preferred_element_type=jnp.float32)
    contrib = lax.dot_general(
        ctx, wo_ref[...], (((1,), (0,)), ((), ())),
        preferred_element_type=jnp.float32)

    rows = pl.ds(qb * QBLK, QBLK)

    @pl.when(h == 0)
    def _():
        partial_ref[rows, :] = contrib

    @pl.when(h != 0)
    def _():
        partial_ref[rows, :] = partial_ref[rows, :] + contrib

    @pl.when(jnp.logical_and(h == pl.num_programs(0) - 1,
                             qb == pl.num_programs(1) - 1))
    def _():
        my = lax.axis_index("i")
        left = lax.rem(my - 1 + N_DEV, N_DEV)
        right = lax.rem(my + 1, N_DEV)

        barrier_sem = pltpu.get_barrier_semaphore()
        for nbr in (left, right):
            pl.semaphore_signal(barrier_sem, inc=1, device_id=(nbr,),
                                device_id_type=pl.DeviceIdType.MESH)
        pl.semaphore_wait(barrier_sem, 2)

        for step in range(N_DEV - 1):
            c_send = lax.rem(my - step + N_DEV, N_DEV)
            rdma = pltpu.make_async_remote_copy(
                src_ref=partial_ref.at[pl.ds(c_send * CHUNK, CHUNK)],
                dst_ref=rs_recv_ref.at[step],
                send_sem=rs_send_sems.at[step],
                recv_sem=rs_recv_sems.at[step],
                device_id=(right,),
                device_id_type=pl.DeviceIdType.MESH,
            )
            rdma.start()
            rdma.wait()
            c_recv = lax.rem(my - step - 1 + N_DEV, N_DEV)
            rrows = pl.ds(c_recv * CHUNK, CHUNK)
            partial_ref[rrows, :] = partial_ref[rrows, :] + rs_recv_ref[step]

        c_own = lax.rem(my + 1, N_DEV)
        orows = pl.ds(c_own * CHUNK, CHUNK)
        out_ref[orows, :] = partial_ref[orows, :]

        for step in range(N_DEV - 1):
            c = lax.rem(my + 1 - step + N_DEV, N_DEV)
            crows = pl.ds(c * CHUNK, CHUNK)
            rdma = pltpu.make_async_remote_copy(
                src_ref=out_ref.at[crows],
                dst_ref=out_ref.at[crows],
                send_sem=ag_send_sems.at[step],
                recv_sem=ag_recv_sems.at[step],
                device_id=(right,),
                device_id_type=pl.DeviceIdType.MESH,
            )
            rdma.start()
            rdma.wait()

        @functools.partial(pl.run_scoped,
                           second_barrier=pltpu.SemaphoreType.REGULAR)
        def _(second_barrier):
            for nbr in (left, right):
                pl.semaphore_signal(second_barrier, inc=1, device_id=(nbr,),
                                    device_id_type=pl.DeviceIdType.MESH)
            pl.semaphore_wait(second_barrier, 2)


def kernel(x, Wq, K_ext, V_ext, Wo):
    my = lax.axis_index("i")
    x2 = x.reshape(SQ, DMODEL)
    wq_s = lax.dynamic_slice(Wq, (0, my * HQ_LOCAL * DH), (DMODEL, HQ_LOCAL * DH))
    wo_s = lax.dynamic_slice(Wo, (my * HQ_LOCAL * DH, 0), (HQ_LOCAL * DH, DMODEL))
    k = K_ext.reshape(SKV, HQ_LOCAL, DH)
    v = V_ext.reshape(SKV, HQ_LOCAL, DH)

    out = pl.pallas_call(
        _body,
        grid=(HQ_LOCAL, N_QB),
        in_specs=[
            pl.BlockSpec((QBLK, DMODEL), lambda h, qb: (qb, 0)),
            pl.BlockSpec((DMODEL, DH), lambda h, qb: (0, h)),
            pl.BlockSpec((SKV, 1, DH), lambda h, qb: (0, h, 0)),
            pl.BlockSpec((SKV, 1, DH), lambda h, qb: (0, h, 0)),
            pl.BlockSpec((DH, DMODEL), lambda h, qb: (h, 0)),
        ],
        out_specs=pl.BlockSpec((SQ, DMODEL), lambda h, qb: (0, 0)),
        out_shape=jax.ShapeDtypeStruct((SQ, DMODEL), jnp.float32),
        scratch_shapes=[
            pltpu.VMEM((SQ, DMODEL), jnp.float32),
            pltpu.VMEM((N_DEV - 1, CHUNK, DMODEL), jnp.float32),
            pltpu.SemaphoreType.DMA((N_DEV - 1,)),
            pltpu.SemaphoreType.DMA((N_DEV - 1,)),
            pltpu.SemaphoreType.DMA((N_DEV - 1,)),
            pltpu.SemaphoreType.DMA((N_DEV - 1,)),
        ],
        compiler_params=pltpu.CompilerParams(
            collective_id=0,
            dimension_semantics=("arbitrary", "arbitrary"),
        ),
    )(x2, wq_s, k, v, wo_s)
    return out.reshape(1, SQ, DMODEL)


# baseline (device time: 304478 ns/iter reference)
import functools

import jax
import jax.numpy as jnp
from jax import lax
from jax.experimental import pallas as pl
from jax.experimental.pallas import tpu as pltpu

N_DEV = 4
SQ = 2048
SKV = 2048
HQ_LOCAL = 8
DH = 128
DMODEL = 1024
QBLK = 512
N_QB = SQ // QBLK
CHUNK = SQ // N_DEV
SCALE = 0.08838834764831843
LOCAL_WINDOW = 128
GLOBAL_TOKENS = 32


def _body(x_ref, wq_ref, k_ref, v_ref, wo_ref, out_ref,
          partial_ref, rs_recv_ref,
          rs_send_sems, rs_recv_sems, ag_send_sems, ag_recv_sems):
    h = pl.program_id(0)
    qb = pl.program_id(1)

    q = lax.dot_general(
        x_ref[...], wq_ref[...],
        (((1,), (0,)), ((), ())), preferred_element_type=jnp.float32)
    k = k_ref[0]
    v = v_ref[0]
    s = lax.dot_general(
        q, k, (((1,), (1,)), ((), ())),
        preferred_element_type=jnp.float32) * SCALE

    qi = qb * QBLK + lax.broadcasted_iota(jnp.int32, (QBLK, SKV), 0)
    ki = lax.broadcasted_iota(jnp.int32, (QBLK, SKV), 1)
    local = jnp.abs(qi - ki) <= LOCAL_WINDOW
    glob = (ki < GLOBAL_TOKENS) | (qi < GLOBAL_TOKENS)
    mask = local | glob
    s = jnp.where(mask, s, -1e9)

    m = jnp.max(s, axis=1, keepdims=True)
    e = jnp.exp(s - m)
    denom = jnp.sum(e, axis=1, keepdims=True)
    w = e / denom

    ctx = lax.dot_general(
        w, v, (((1,), (0,)), ((), ())), preferred_element_type=jnp.float32)
    contrib = lax.dot_general(
        ctx, wo_ref[...], (((1,), (0,)), ((), ())),
        preferred_element_type=jnp.float32)

    rows = pl.ds(qb * QBLK, QBLK)

    @pl.when(h == 0)
    def _():
        partial_ref[rows, :] = contrib

    @pl.when(h != 0)
    def _():
        partial_ref[rows, :] = partial_ref[rows, :] + contrib

    @pl.when(jnp.logical_and(h == pl.num_programs(0) - 1,
                             qb == pl.num_programs(1) - 1))
    def _():
        my = lax.axis_index("i")
        left = lax.rem(my - 1 + N_DEV, N_DEV)
        right = lax.rem(my + 1, N_DEV)

        barrier_sem = pltpu.get_barrier_semaphore()
        for nbr in (left, right):
            pl.semaphore_signal(barrier_sem, inc=1, device_id=(nbr,),
                                device_id_type=pl.DeviceIdType.MESH)
        pl.semaphore_wait(barrier_sem, 2)

        for step in range(N_DEV - 1):
            c_send = lax.rem(my - step + N_DEV, N_DEV)
            rdma = pltpu.make_async_remote_copy(
                src_ref=partial_ref.at[pl.ds(c_send * CHUNK, CHUNK)],
                dst_ref=rs_recv_ref.at[step],
                send_sem=rs_send_sems.at[step],
                recv_sem=rs_recv_sems.at[step],
                device_id=(right,),
                device_id_type=pl.DeviceIdType.MESH,
            )
            rdma.start()
            rdma.wait()
            c_recv = lax.rem(my - step - 1 + N_DEV, N_DEV)
            rrows = pl.ds(c_recv * CHUNK, CHUNK)
            partial_ref[rrows, :] = partial_ref[rrows, :] + rs_recv_ref[step]

        c_own = lax.rem(my + 1, N_DEV)
        orows = pl.ds(c_own * CHUNK, CHUNK)
        out_ref[orows, :] = partial_ref[orows, :]

        for step in range(N_DEV - 1):
            c = lax.rem(my + 1 - step + N_DEV, N_DEV)
            crows = pl.ds(c * CHUNK, CHUNK)
            rdma = pltpu.make_async_remote_copy(
                src_ref=out_ref.at[crows],
                dst_ref=out_ref.at[crows],
                send_sem=ag_send_sems.at[step],
                recv_sem=ag_recv_sems.at[step],
                device_id=(right,),
                device_id_type=pl.DeviceIdType.MESH,
            )
            rdma.start()
            rdma.wait()

        @functools.partial(pl.run_scoped,
                           second_barrier=pltpu.SemaphoreType.REGULAR)
        def _(second_barrier):
            for nbr in (left, right):
                pl.semaphore_signal(second_barrier, inc=1, device_id=(nbr,),
                                    device_id_type=pl.DeviceIdType.MESH)
            pl.semaphore_wait(second_barrier, 2)


def kernel(x, Wq, K_ext, V_ext, Wo):
    my = lax.axis_index("i")
    x2 = x.reshape(SQ, DMODEL)
    wq_s = lax.dynamic_slice(Wq, (0, my * HQ_LOCAL * DH), (DMODEL, HQ_LOCAL * DH))
    wo_s = lax.dynamic_slice(Wo, (my * HQ_LOCAL * DH, 0), (HQ_LOCAL * DH, DMODEL))
    k = jnp.transpose(K_ext.reshape(SKV, HQ_LOCAL, DH), (1, 0, 2))
    v = jnp.transpose(V_ext.reshape(SKV, HQ_LOCAL, DH), (1, 0, 2))

    out = pl.pallas_call(
        _body,
        grid=(HQ_LOCAL, N_QB),
        in_specs=[
            pl.BlockSpec((QBLK, DMODEL), lambda h, qb: (qb, 0)),
            pl.BlockSpec((DMODEL, DH), lambda h, qb: (0, h)),
            pl.BlockSpec((1, SKV, DH), lambda h, qb: (h, 0, 0)),
            pl.BlockSpec((1, SKV, DH), lambda h, qb: (h, 0, 0)),
            pl.BlockSpec((DH, DMODEL), lambda h, qb: (h, 0)),
        ],
        out_specs=pl.BlockSpec((SQ, DMODEL), lambda h, qb: (0, 0)),
        out_shape=jax.ShapeDtypeStruct((SQ, DMODEL), jnp.float32),
        scratch_shapes=[
            pltpu.VMEM((SQ, DMODEL), jnp.float32),
            pltpu.VMEM((N_DEV - 1, CHUNK, DMODEL), jnp.float32),
            pltpu.SemaphoreType.DMA((N_DEV - 1,)),
            pltpu.SemaphoreType.DMA((N_DEV - 1,)),
            pltpu.SemaphoreType.DMA((N_DEV - 1,)),
            pltpu.SemaphoreType.DMA((N_DEV - 1,)),
        ],
        compiler_params=pltpu.CompilerParams(
            collective_id=0,
            dimension_semantics=("arbitrary", "arbitrary"),
        ),
    )(x2, wq_s, k, v, wo_s)
    return out.reshape(1, SQ, DMODEL)


# device time: 212623 ns/iter; 1.4320x vs baseline; 1.4320x over previous
import functools

import jax
import jax.numpy as jnp
from jax import lax
from jax.experimental import pallas as pl
from jax.experimental.pallas import tpu as pltpu

N_DEV = 4
SQ = 2048
SKV = 2048
HQ_LOCAL = 8
DH = 128
DMODEL = 1024
QBLK = 512
N_QB = SQ // QBLK
CHUNK = SQ // N_DEV
SCALE = 0.08838834764831843
LOCAL_WINDOW = 128
GLOBAL_TOKENS = 32


def _chunk_rows(c):
    return pl.ds(c * CHUNK, CHUNK)


def _body(x_ref, wq_ref, k_ref, v_ref, wo_ref, out_ref,
          partial_ref, rs_recv_ref,
          rs_send_sems, rs_recv_sems, ag_send_sems, ag_recv_sems):
    j = pl.program_id(0)
    h = pl.program_id(1)
    my = lax.axis_index("i")
    left = lax.rem(my + N_DEV - 1, N_DEV)
    right = lax.rem(my + 1, N_DEV)
    qb = lax.rem(my - j + 2 * N_DEV, N_DEV)
    rows = _chunk_rows(qb)

    q = lax.dot_general(
        x_ref[rows, :], wq_ref[...],
        (((1,), (0,)), ((), ())), preferred_element_type=jnp.float32)
    k = k_ref[0]
    v = v_ref[0]
    s = lax.dot_general(
        q, k, (((1,), (1,)), ((), ())),
        preferred_element_type=jnp.float32) * SCALE

    qi = qb * QBLK + lax.broadcasted_iota(jnp.int32, (QBLK, SKV), 0)
    ki = lax.broadcasted_iota(jnp.int32, (QBLK, SKV), 1)
    mask = (jnp.abs(qi - ki) <= LOCAL_WINDOW) | (ki < GLOBAL_TOKENS) | (
        qi < GLOBAL_TOKENS)
    s = jnp.where(mask, s, -1e9)

    m = jnp.max(s, axis=1, keepdims=True)
    e = jnp.exp(s - m)
    w = e / jnp.sum(e, axis=1, keepdims=True)

    ctx = lax.dot_general(
        w, v, (((1,), (0,)), ((), ())), preferred_element_type=jnp.float32)
    contrib = lax.dot_general(
        ctx, wo_ref[...], (((1,), (0,)), ((), ())),
        preferred_element_type=jnp.float32)

    @pl.when(h == 0)
    def _():
        partial_ref[rows, :] = contrib

    @pl.when(h != 0)
    def _():
        partial_ref[rows, :] = partial_ref[rows, :] + contrib

    def rs_send(step):
        c = lax.rem(my - step + 2 * N_DEV, N_DEV)
        rdma = pltpu.make_async_remote_copy(
            src_ref=partial_ref.at[_chunk_rows(c)],
            dst_ref=rs_recv_ref.at[step],
            send_sem=rs_send_sems.at[step],
            recv_sem=rs_recv_sems.at[step],
            device_id=(right,),
            device_id_type=pl.DeviceIdType.MESH,
        )
        rdma.start()
        return rdma

    def rs_wait_and_add(step):
        c = lax.rem(my - step - 1 + 2 * N_DEV, N_DEV)
        rdma = pltpu.make_async_remote_copy(
            src_ref=partial_ref.at[_chunk_rows(c)],
            dst_ref=rs_recv_ref.at[step],
            send_sem=rs_send_sems.at[step],
            recv_sem=rs_recv_sems.at[step],
            device_id=(right,),
            device_id_type=pl.DeviceIdType.MESH,
        )
        rdma.wait_recv()
        rrows = _chunk_rows(c)
        partial_ref[rrows, :] = partial_ref[rrows, :] + rs_recv_ref[step]

    def rs_wait_send(step):
        c = lax.rem(my - step + 2 * N_DEV, N_DEV)
        rdma = pltpu.make_async_remote_copy(
            src_ref=partial_ref.at[_chunk_rows(c)],
            dst_ref=rs_recv_ref.at[step],
            send_sem=rs_send_sems.at[step],
            recv_sem=rs_recv_sems.at[step],
            device_id=(right,),
            device_id_type=pl.DeviceIdType.MESH,
        )
        rdma.wait_send()

    def ag_copy(c, sem_idx, target):
        return pltpu.make_async_remote_copy(
            src_ref=out_ref.at[_chunk_rows(c)],
            dst_ref=out_ref.at[_chunk_rows(c)],
            send_sem=ag_send_sems.at[sem_idx],
            recv_sem=ag_recv_sems.at[sem_idx],
            device_id=(target,),
            device_id_type=pl.DeviceIdType.MESH,
        )

    last_h = h == HQ_LOCAL - 1

    @pl.when(jnp.logical_and(last_h, j == 0))
    def _():
        barrier_sem = pltpu.get_barrier_semaphore()
        for nbr in (left, right):
            pl.semaphore_signal(barrier_sem, inc=1, device_id=(nbr,),
                                device_id_type=pl.DeviceIdType.MESH)
        pl.semaphore_wait(barrier_sem, 2)
        rs_send(0)

    @pl.when(jnp.logical_and(last_h, jnp.logical_and(j > 0, j < N_QB - 1)))
    def _():
        for step in range(N_QB - 2):
            @pl.when(j == step + 1)
            def _():
                rs_wait_and_add(step)
                rs_wait_send(step)
                rs_send(step + 1)

    @pl.when(jnp.logical_and(last_h, j == N_QB - 1))
    def _():
        rs_wait_and_add(N_DEV - 2)
        rs_wait_send(N_DEV - 2)

        c_own = lax.rem(my + 1, N_DEV)
        orows = _chunk_rows(c_own)
        out_ref[orows, :] = partial_ref[orows, :]

        send_r = ag_copy(c_own, 0, right)
        send_l = ag_copy(c_own, 1, left)
        send_r.start()
        send_l.start()

        ag_copy(my, 0, right).wait_recv()
        fwd = ag_copy(my, 2, right)
        fwd.start()

        ag_copy(lax.rem(my + 2, N_DEV), 1, right).wait_recv()
        ag_copy(left, 2, right).wait_recv()

        send_r.wait_send()
        send_l.wait_send()
        fwd.wait_send()

        @functools.partial(pl.run_scoped,
                           second_barrier=pltpu.SemaphoreType.REGULAR)
        def _(second_barrier):
            for nbr in (left, right):
                pl.semaphore_signal(second_barrier, inc=1, device_id=(nbr,),
                                    device_id_type=pl.DeviceIdType.MESH)
            pl.semaphore_wait(second_barrier, 2)


def kernel(x, Wq, K_ext, V_ext, Wo):
    my = lax.axis_index("i")
    x2 = x.reshape(SQ, DMODEL)
    wq_s = lax.dynamic_slice(Wq, (0, my * HQ_LOCAL * DH), (DMODEL, HQ_LOCAL * DH))
    wo_s = lax.dynamic_slice(Wo, (my * HQ_LOCAL * DH, 0), (HQ_LOCAL * DH, DMODEL))
    k = jnp.transpose(K_ext.reshape(SKV, HQ_LOCAL, DH), (1, 0, 2))
    v = jnp.transpose(V_ext.reshape(SKV, HQ_LOCAL, DH), (1, 0, 2))

    out = pl.pallas_call(
        _body,
        grid=(N_QB, HQ_LOCAL),
        in_specs=[
            pl.BlockSpec((SQ, DMODEL), lambda j, h: (0, 0)),
            pl.BlockSpec((DMODEL, DH), lambda j, h: (0, h)),
            pl.BlockSpec((1, SKV, DH), lambda j, h: (h, 0, 0)),
            pl.BlockSpec((1, SKV, DH), lambda j, h: (h, 0, 0)),
            pl.BlockSpec((DH, DMODEL), lambda j, h: (h, 0)),
        ],
        out_specs=pl.BlockSpec((SQ, DMODEL), lambda j, h: (0, 0)),
        out_shape=jax.ShapeDtypeStruct((SQ, DMODEL), jnp.float32),
        scratch_shapes=[
            pltpu.VMEM((SQ, DMODEL), jnp.float32),
            pltpu.VMEM((N_DEV - 1, CHUNK, DMODEL), jnp.float32),
            pltpu.SemaphoreType.DMA((N_DEV - 1,)),
            pltpu.SemaphoreType.DMA((N_DEV - 1,)),
            pltpu.SemaphoreType.DMA((N_DEV - 1,)),
            pltpu.SemaphoreType.DMA((N_DEV - 1,)),
        ],
        compiler_params=pltpu.CompilerParams(
            collective_id=0,
            dimension_semantics=("arbitrary", "arbitrary"),
        ),
    )(x2, wq_s, k, v, wo_s)
    return out.reshape(1, SQ, DMODEL)
